# baseline (speedup 1.0000x reference)
import functools

import jax
import jax.numpy as jnp
from jax.experimental import pallas as pl
from jax.experimental.pallas import tpu as pltpu

NUM_EXPERTS = 16
TOP_K = 2
HIDDEN_DIM = 2048
NUM_TOKENS = 16384

BLOCK_N = 512
NBUF = 8


def _router_block(x_hbm, c_ref, idx_ref, tkp_ref, probs_ref, xbuf, sems):
    i = pl.program_id(0)
    nchunk = pl.num_programs(0)

    def start(chunk):
        slot = jax.lax.rem(chunk, NBUF)
        pltpu.make_async_copy(
            x_hbm.at[pl.ds(chunk * BLOCK_N, BLOCK_N), :],
            xbuf.at[slot],
            sems.at[slot],
        ).start()

    @pl.when(i == 0)
    def _prologue():
        for j in range(NBUF - 1):
            start(j)

    @pl.when(i + NBUF - 1 < nchunk)
    def _steady():
        start(i + NBUF - 1)

    slot = jax.lax.rem(i, NBUF)
    pltpu.make_async_copy(
        x_hbm.at[pl.ds(i * BLOCK_N, BLOCK_N), :],
        xbuf.at[slot],
        sems.at[slot],
    ).wait()

    x = xbuf[slot]
    c = c_ref[...]
    b = x.shape[0]

    x2 = jnp.sum(x * x, axis=1, keepdims=True)
    c2 = jnp.sum(c * c, axis=1)[None, :]
    xc = jax.lax.dot_general(
        x, c, (((1,), (1,)), ((), ())),
        preferred_element_type=jnp.float32)
    d2 = jnp.maximum(x2 + c2 - 2.0 * xc, 0.0)
    neg_d = -jnp.sqrt(d2)

    m = jnp.max(neg_d, axis=1, keepdims=True)
    e = jnp.exp(neg_d - m)
    s = jnp.sum(e, axis=1, keepdims=True)
    probs = e / s
    probs_ref[...] = probs

    lane = jax.lax.broadcasted_iota(jnp.int32, (b, NUM_EXPERTS), 1)
    m1 = jnp.max(probs, axis=1, keepdims=True)
    i1 = jnp.min(jnp.where(probs == m1, lane, NUM_EXPERTS),
                 axis=1, keepdims=True)
    masked = jnp.where(lane == i1, -jnp.float32(1.0), probs)
    m2 = jnp.max(masked, axis=1, keepdims=True)
    i2 = jnp.min(jnp.where(masked == m2, lane, NUM_EXPERTS),
                 axis=1, keepdims=True)

    denom = m1 + m2
    idx_ref[...] = jnp.concatenate([i1, i2], axis=1)
    tkp_ref[...] = jnp.concatenate([m1 / denom, m2 / denom], axis=1)


@jax.jit
def kernel(x, centroids):
    n, d = x.shape
    e = centroids.shape[0]
    grid = (n // BLOCK_N,)
    out_shapes = (
        jax.ShapeDtypeStruct((n, TOP_K), jnp.int32),
        jax.ShapeDtypeStruct((n, TOP_K), jnp.float32),
        jax.ShapeDtypeStruct((n, e), jnp.float32),
    )
    return pl.pallas_call(
        _router_block,
        grid=grid,
        in_specs=[
            pl.BlockSpec(memory_space=pltpu.MemorySpace.HBM),
            pl.BlockSpec((e, d), lambda i: (0, 0)),
        ],
        out_specs=(
            pl.BlockSpec((BLOCK_N, TOP_K), lambda i: (i, 0)),
            pl.BlockSpec((BLOCK_N, TOP_K), lambda i: (i, 0)),
            pl.BlockSpec((BLOCK_N, e), lambda i: (i, 0)),
        ),
        out_shape=out_shapes,
        scratch_shapes=[
            pltpu.VMEM((NBUF, BLOCK_N, d), jnp.float32),
            pltpu.SemaphoreType.DMA((NBUF,)),
        ],
        compiler_params=pltpu.CompilerParams(
            dimension_semantics=("arbitrary",),
        ),
    )(x, centroids)


# speedup vs baseline: 1.0494x; 1.0494x over previous
import functools

import jax
import jax.numpy as jnp
from jax.experimental import pallas as pl
from jax.experimental.pallas import tpu as pltpu

NUM_EXPERTS = 16
TOP_K = 2
HIDDEN_DIM = 2048
NUM_TOKENS = 16384

BLOCK_N = 1024
NBUF = 6


def _router_block(x_hbm, c_ref, idx_ref, tkp_ref, probs_ref, xbuf, sems):
    i = pl.program_id(0)
    nchunk = pl.num_programs(0)

    def start(chunk):
        slot = jax.lax.rem(chunk, NBUF)
        pltpu.make_async_copy(
            x_hbm.at[pl.ds(chunk * BLOCK_N, BLOCK_N), :],
            xbuf.at[slot],
            sems.at[slot],
        ).start()

    @pl.when(i == 0)
    def _prologue():
        for j in range(NBUF - 1):
            start(j)

    @pl.when(i + NBUF - 1 < nchunk)
    def _steady():
        start(i + NBUF - 1)

    slot = jax.lax.rem(i, NBUF)
    pltpu.make_async_copy(
        x_hbm.at[pl.ds(i * BLOCK_N, BLOCK_N), :],
        xbuf.at[slot],
        sems.at[slot],
    ).wait()

    x = xbuf[slot]
    c = c_ref[...]
    b = x.shape[0]

    x2 = jnp.sum(x * x, axis=1, keepdims=True)
    c2 = jnp.sum(c * c, axis=1)[None, :]
    xc = jax.lax.dot_general(
        x, c, (((1,), (1,)), ((), ())),
        preferred_element_type=jnp.float32)
    d2 = jnp.maximum(x2 + c2 - 2.0 * xc, 0.0)
    neg_d = -jnp.sqrt(d2)

    m = jnp.max(neg_d, axis=1, keepdims=True)
    e = jnp.exp(neg_d - m)
    s = jnp.sum(e, axis=1, keepdims=True)
    probs = e / s
    probs_ref[...] = probs

    lane = jax.lax.broadcasted_iota(jnp.int32, (b, NUM_EXPERTS), 1)
    m1 = jnp.max(probs, axis=1, keepdims=True)
    i1 = jnp.min(jnp.where(probs == m1, lane, NUM_EXPERTS),
                 axis=1, keepdims=True)
    masked = jnp.where(lane == i1, -jnp.float32(1.0), probs)
    m2 = jnp.max(masked, axis=1, keepdims=True)
    i2 = jnp.min(jnp.where(masked == m2, lane, NUM_EXPERTS),
                 axis=1, keepdims=True)

    denom = m1 + m2
    idx_ref[...] = jnp.concatenate([i1, i2], axis=1)
    tkp_ref[...] = jnp.concatenate([m1 / denom, m2 / denom], axis=1)


@jax.jit
def kernel(x, centroids):
    n, d = x.shape
    e = centroids.shape[0]
    grid = (n // BLOCK_N,)
    out_shapes = (
        jax.ShapeDtypeStruct((n, TOP_K), jnp.int32),
        jax.ShapeDtypeStruct((n, TOP_K), jnp.float32),
        jax.ShapeDtypeStruct((n, e), jnp.float32),
    )
    return pl.pallas_call(
        _router_block,
        grid=grid,
        in_specs=[
            pl.BlockSpec(memory_space=pltpu.MemorySpace.HBM),
            pl.BlockSpec((e, d), lambda i: (0, 0)),
        ],
        out_specs=(
            pl.BlockSpec((BLOCK_N, TOP_K), lambda i: (i, 0)),
            pl.BlockSpec((BLOCK_N, TOP_K), lambda i: (i, 0)),
            pl.BlockSpec((BLOCK_N, e), lambda i: (i, 0)),
        ),
        out_shape=out_shapes,
        scratch_shapes=[
            pltpu.VMEM((NBUF, BLOCK_N, d), jnp.float32),
            pltpu.SemaphoreType.DMA((NBUF,)),
        ],
        compiler_params=pltpu.CompilerParams(
            dimension_semantics=("arbitrary",),
        ),
    )(x, centroids)
